# baseline (device time: 102236 ns/iter reference)
import jax
import jax.numpy as jnp
from jax import lax
from jax.experimental import pallas as pl
from jax.experimental.pallas import tpu as pltpu

N_DEV = 8
B = 2
SQ = 256
SKV = 256
HQ_LOC = 4
DH = 64
D_MODEL = 512
BLK = 64
BFLY_BITS = (4, 3, 1)


def kernel(x, Wq, K_ext, V_ext, Wo):
    def body(x_ref, wq_ref, k_ref, v_ref, wo_ref, out_ref,
             kv_buf, kv_f32, stage_ref, ctx_buf, bfly_snd, bfly_buf,
             scat_send_sems, scat_recv_sems, bfly_send_sems, bfly_recv_sems,
             local_sems):
        my_pos = lax.axis_index("i")

        def scatter_rdmas():
            rdmas = []
            for p in range(1, N_DEV):
                for idx in range(2):
                    rdmas.append(pltpu.make_async_remote_copy(
                        src_ref=stage_ref.at[p - 1, idx],
                        dst_ref=kv_buf.at[idx],
                        send_sem=scat_send_sems.at[p - 1, idx],
                        recv_sem=scat_recv_sems.at[idx],
                        device_id=(p,),
                        device_id_type=pl.DeviceIdType.MESH,
                    ))
            return rdmas

        with jax.named_scope("stage_and_send"):
            @pl.when(my_pos == 0)
            def _():
                cps = [
                    pltpu.make_async_copy(k_ref, kv_f32.at[0], local_sems.at[0]),
                    pltpu.make_async_copy(v_ref, kv_f32.at[1], local_sems.at[1]),
                ]
                for cp in cps:
                    cp.start()
                for cp in cps:
                    cp.wait()
                for p in range(1, N_DEV):
                    lo = HQ_LOC * p
                    stage_ref[p - 1, 0] = kv_f32[0, :, :, lo:lo + HQ_LOC, :].astype(
                        jnp.bfloat16)
                    stage_ref[p - 1, 1] = kv_f32[1, :, :, lo:lo + HQ_LOC, :].astype(
                        jnp.bfloat16)
                for r in scatter_rdmas():
                    r.start()
                kv_buf[0] = kv_f32[0, :, :, 0:HQ_LOC, :].astype(jnp.bfloat16)
                kv_buf[1] = kv_f32[1, :, :, 0:HQ_LOC, :].astype(jnp.bfloat16)

        with jax.named_scope("q_proj"):
            q_all = [jnp.dot(x_ref[b], wq_ref[...],
                             preferred_element_type=jnp.float32)
                     for b in range(B)]

        with jax.named_scope("wait_kv"):
            @pl.when(my_pos != 0)
            def _():
                for idx in range(2):
                    recv = pltpu.make_async_remote_copy(
                        src_ref=kv_buf.at[idx], dst_ref=kv_buf.at[idx],
                        send_sem=scat_send_sems.at[0, idx],
                        recv_sem=scat_recv_sems.at[idx],
                        device_id=(0,), device_id_type=pl.DeviceIdType.MESH,
                    )
                    recv.wait_recv()

        with jax.named_scope("attn"):
            row = lax.broadcasted_iota(jnp.int32, (SQ, SKV), 0) // BLK
            col = lax.broadcasted_iota(jnp.int32, (SQ, SKV), 1) // BLK
            mask = col <= row
            for b in range(B):
                for h in range(HQ_LOC):
                    q_bh = q_all[b][:, h * DH:(h + 1) * DH].astype(jnp.bfloat16)
                    k_bh = kv_buf[0, b, :, h, :]
                    v_bh = kv_buf[1, b, :, h, :]
                    s = lax.dot_general(
                        q_bh, k_bh, (((1,), (1,)), ((), ())),
                        preferred_element_type=jnp.float32) * 0.125
                    s = jnp.where(mask, s, -1e9)
                    m = jnp.max(s, axis=1, keepdims=True)
                    w = jnp.exp(s - m)
                    w = w / jnp.sum(w, axis=1, keepdims=True)
                    ctx_buf[b, :, h * DH:(h + 1) * DH] = jnp.dot(
                        w.astype(jnp.bfloat16), v_bh,
                        preferred_element_type=jnp.float32)

        with jax.named_scope("o_proj"):
            for b in range(B):
                out_ref[b] = jnp.dot(ctx_buf[b], wo_ref[...],
                                     preferred_element_type=jnp.float32)

        for s_i, bit in enumerate(BFLY_BITS):
            with jax.named_scope(f"bfly#stage={s_i}"):
                partner = my_pos ^ bit
                bfly_snd[...] = out_ref[...].astype(jnp.bfloat16)
                rdma = pltpu.make_async_remote_copy(
                    src_ref=bfly_snd,
                    dst_ref=bfly_buf.at[s_i],
                    send_sem=bfly_send_sems.at[s_i],
                    recv_sem=bfly_recv_sems.at[s_i],
                    device_id=(partner,),
                    device_id_type=pl.DeviceIdType.MESH,
                )
                rdma.start()
                rdma.wait()
                out_ref[...] = out_ref[...] + bfly_buf[s_i].astype(jnp.float32)

        with jax.named_scope("drain_sends"):
            @pl.when(my_pos == 0)
            def _():
                for r in scatter_rdmas():
                    r.wait_send()

    return pl.pallas_call(
        body,
        out_shape=jax.ShapeDtypeStruct((B, SQ, D_MODEL), jnp.float32),
        in_specs=[
            pl.BlockSpec(memory_space=pltpu.VMEM),
            pl.BlockSpec(memory_space=pltpu.VMEM),
            pl.BlockSpec(memory_space=pltpu.MemorySpace.HBM),
            pl.BlockSpec(memory_space=pltpu.MemorySpace.HBM),
            pl.BlockSpec(memory_space=pltpu.VMEM),
        ],
        out_specs=pl.BlockSpec(memory_space=pltpu.VMEM),
        scratch_shapes=[
            pltpu.VMEM((2, B, SKV, HQ_LOC, DH), jnp.bfloat16),
            pltpu.VMEM((2, B, SKV, N_DEV * HQ_LOC, DH), jnp.float32),
            pltpu.VMEM((N_DEV - 1, 2, B, SKV, HQ_LOC, DH), jnp.bfloat16),
            pltpu.VMEM((B, SQ, HQ_LOC * DH), jnp.float32),
            pltpu.VMEM((B, SQ, D_MODEL), jnp.bfloat16),
            pltpu.VMEM((3, B, SQ, D_MODEL), jnp.bfloat16),
            pltpu.SemaphoreType.DMA((N_DEV - 1, 2)),
            pltpu.SemaphoreType.DMA((2,)),
            pltpu.SemaphoreType.DMA((3,)),
            pltpu.SemaphoreType.DMA((3,)),
            pltpu.SemaphoreType.DMA((2,)),
        ],
    )(x, Wq, K_ext, V_ext, Wo)


# device time: 65724 ns/iter; 1.5555x vs baseline; 1.5555x over previous
import jax
import jax.numpy as jnp
from jax import lax
from jax.experimental import pallas as pl
from jax.experimental.pallas import tpu as pltpu

N_DEV = 8
B = 2
SQ = 256
SKV = 256
HQ_LOC = 4
DH = 64
D_MODEL = 512
BLK = 64
BFLY_BITS = (4, 3, 1)


def kernel(x, Wq, K_ext, V_ext, Wo):
    K_t = jnp.transpose(K_ext, (0, 2, 3, 1))
    V_t = jnp.transpose(V_ext, (0, 2, 3, 1))

    def body(x_ref, wq_ref, k_ref, v_ref, wo_ref, out_ref,
             kv_buf, kv_f32, stage_ref, ctx_buf, bfly_snd, bfly_buf,
             scat_send_sems, scat_recv_sems, bfly_send_sems, bfly_recv_sems,
             local_sems):
        my_pos = lax.axis_index("i")

        def scatter_rdmas():
            rdmas = []
            for p in range(1, N_DEV):
                for idx in range(2):
                    rdmas.append(pltpu.make_async_remote_copy(
                        src_ref=stage_ref.at[p - 1, idx],
                        dst_ref=kv_buf.at[idx],
                        send_sem=scat_send_sems.at[p - 1, idx],
                        recv_sem=scat_recv_sems.at[idx],
                        device_id=(p,),
                        device_id_type=pl.DeviceIdType.MESH,
                    ))
            return rdmas

        @pl.when(my_pos == 0)
        def _():
            cps = [
                pltpu.make_async_copy(k_ref, kv_f32.at[0], local_sems.at[0]),
                pltpu.make_async_copy(v_ref, kv_f32.at[1], local_sems.at[1]),
            ]
            for cp in cps:
                cp.start()
            for cp in cps:
                cp.wait()
            for p in range(1, N_DEV):
                lo = HQ_LOC * p
                stage_ref[p - 1, 0] = kv_f32[0, :, lo:lo + HQ_LOC, :, :].astype(
                    jnp.bfloat16)
                stage_ref[p - 1, 1] = kv_f32[1, :, lo:lo + HQ_LOC, :, :].astype(
                    jnp.bfloat16)
            for r in scatter_rdmas():
                r.start()
            kv_buf[0] = kv_f32[0, :, 0:HQ_LOC, :, :].astype(jnp.bfloat16)
            kv_buf[1] = kv_f32[1, :, 0:HQ_LOC, :, :].astype(jnp.bfloat16)

        q_all = [jnp.dot(x_ref[b], wq_ref[...],
                         preferred_element_type=jnp.float32) for b in range(B)]

        @pl.when(my_pos != 0)
        def _():
            for idx in range(2):
                recv = pltpu.make_async_remote_copy(
                    src_ref=kv_buf.at[idx], dst_ref=kv_buf.at[idx],
                    send_sem=scat_send_sems.at[0, idx],
                    recv_sem=scat_recv_sems.at[idx],
                    device_id=(0,), device_id_type=pl.DeviceIdType.MESH,
                )
                recv.wait_recv()

        row = lax.broadcasted_iota(jnp.int32, (SQ, SKV), 0) // BLK
        col = lax.broadcasted_iota(jnp.int32, (SQ, SKV), 1) // BLK
        mask = col <= row
        for b in range(B):
            for h in range(HQ_LOC):
                q_bh = q_all[b][:, h * DH:(h + 1) * DH].astype(jnp.bfloat16)
                k_bh = kv_buf[0, b, h]
                v_bh = kv_buf[1, b, h]
                s = lax.dot_general(
                    q_bh, k_bh, (((1,), (0,)), ((), ())),
                    preferred_element_type=jnp.float32) * 0.125
                s = jnp.where(mask, s, -1e9)
                m = jnp.max(s, axis=1, keepdims=True)
                w = jnp.exp(s - m)
                w = w / jnp.sum(w, axis=1, keepdims=True)
                ctx_buf[b, :, h * DH:(h + 1) * DH] = lax.dot_general(
                    w.astype(jnp.bfloat16), v_bh, (((1,), (1,)), ((), ())),
                    preferred_element_type=jnp.float32)

        for b in range(B):
            out_ref[b] = jnp.dot(ctx_buf[b], wo_ref[...],
                                 preferred_element_type=jnp.float32)

        for s_i, bit in enumerate(BFLY_BITS):
            partner = my_pos ^ bit
            bfly_snd[...] = out_ref[...].astype(jnp.bfloat16)
            rdma = pltpu.make_async_remote_copy(
                src_ref=bfly_snd,
                dst_ref=bfly_buf.at[s_i],
                send_sem=bfly_send_sems.at[s_i],
                recv_sem=bfly_recv_sems.at[s_i],
                device_id=(partner,),
                device_id_type=pl.DeviceIdType.MESH,
            )
            rdma.start()
            rdma.wait()
            out_ref[...] = out_ref[...] + bfly_buf[s_i].astype(jnp.float32)

        @pl.when(my_pos == 0)
        def _():
            for r in scatter_rdmas():
                r.wait_send()

    return pl.pallas_call(
        body,
        out_shape=jax.ShapeDtypeStruct((B, SQ, D_MODEL), jnp.float32),
        in_specs=[
            pl.BlockSpec(memory_space=pltpu.MemorySpace.VMEM),
            pl.BlockSpec(memory_space=pltpu.MemorySpace.VMEM),
            pl.BlockSpec(memory_space=pltpu.MemorySpace.HBM),
            pl.BlockSpec(memory_space=pltpu.MemorySpace.HBM),
            pl.BlockSpec(memory_space=pltpu.MemorySpace.VMEM),
        ],
        out_specs=pl.BlockSpec(memory_space=pltpu.MemorySpace.VMEM),
        scratch_shapes=[
            pltpu.VMEM((2, B, HQ_LOC, DH, SKV), jnp.bfloat16),
            pltpu.VMEM((2, B, N_DEV * HQ_LOC, DH, SKV), jnp.float32),
            pltpu.VMEM((N_DEV - 1, 2, B, HQ_LOC, DH, SKV), jnp.bfloat16),
            pltpu.VMEM((B, SQ, HQ_LOC * DH), jnp.float32),
            pltpu.VMEM((B, SQ, D_MODEL), jnp.bfloat16),
            pltpu.VMEM((3, B, SQ, D_MODEL), jnp.bfloat16),
            pltpu.SemaphoreType.DMA((N_DEV - 1, 2)),
            pltpu.SemaphoreType.DMA((2,)),
            pltpu.SemaphoreType.DMA((3,)),
            pltpu.SemaphoreType.DMA((3,)),
            pltpu.SemaphoreType.DMA((2,)),
        ],
    )(x, Wq, K_t, V_t, Wo)


# device time: 65474 ns/iter; 1.5615x vs baseline; 1.0038x over previous
import jax
import jax.numpy as jnp
from jax import lax
from jax.experimental import pallas as pl
from jax.experimental.pallas import tpu as pltpu

N_DEV = 8
B = 2
SQ = 256
SKV = 256
HQ_LOC = 4
DH = 64
D_MODEL = 512
BLK = 64
BFLY_BITS = (4, 3, 1)


def kernel(x, Wq, K_ext, V_ext, Wo):
    K_t = jnp.transpose(K_ext, (0, 2, 3, 1))
    V_t = jnp.transpose(V_ext, (0, 2, 3, 1))

    def body(x_ref, wq_ref, k_ref, v_ref, wo_ref, out_ref,
             x_v, wq_v, wo_v, acc, kv_buf, kv_f32, stage_ref, ctx_buf,
             bfly_snd, bfly_buf,
             scat_send_sems, scat_recv_sems, bfly_send_sems, bfly_recv_sems,
             local_sems):
        my_pos = lax.axis_index("i")

        in_cps = [
            pltpu.make_async_copy(x_ref, x_v, local_sems.at[2]),
            pltpu.make_async_copy(wq_ref, wq_v, local_sems.at[3]),
            pltpu.make_async_copy(wo_ref, wo_v, local_sems.at[4]),
        ]
        for cp in in_cps:
            cp.start()

        def scatter_rdmas():
            rdmas = []
            for p in range(1, N_DEV):
                for idx in range(2):
                    rdmas.append(pltpu.make_async_remote_copy(
                        src_ref=stage_ref.at[p - 1, idx],
                        dst_ref=kv_buf.at[idx],
                        send_sem=scat_send_sems.at[p - 1, idx],
                        recv_sem=scat_recv_sems.at[idx],
                        device_id=(p,),
                        device_id_type=pl.DeviceIdType.MESH,
                    ))
            return rdmas

        @pl.when(my_pos == 0)
        def _():
            cps = [
                pltpu.make_async_copy(k_ref, kv_f32.at[0], local_sems.at[0]),
                pltpu.make_async_copy(v_ref, kv_f32.at[1], local_sems.at[1]),
            ]
            for cp in cps:
                cp.start()
            for cp in cps:
                cp.wait()
            for p in range(1, N_DEV):
                lo = HQ_LOC * p
                stage_ref[p - 1, 0] = kv_f32[0, :, lo:lo + HQ_LOC, :, :].astype(
                    jnp.bfloat16)
                stage_ref[p - 1, 1] = kv_f32[1, :, lo:lo + HQ_LOC, :, :].astype(
                    jnp.bfloat16)
            for r in scatter_rdmas():
                r.start()
            kv_buf[0] = kv_f32[0, :, 0:HQ_LOC, :, :].astype(jnp.bfloat16)
            kv_buf[1] = kv_f32[1, :, 0:HQ_LOC, :, :].astype(jnp.bfloat16)

        for cp in in_cps:
            cp.wait()
        q_all = [jnp.dot(x_v[b], wq_v[...],
                         preferred_element_type=jnp.float32) for b in range(B)]

        @pl.when(my_pos != 0)
        def _():
            for idx in range(2):
                recv = pltpu.make_async_remote_copy(
                    src_ref=kv_buf.at[idx], dst_ref=kv_buf.at[idx],
                    send_sem=scat_send_sems.at[0, idx],
                    recv_sem=scat_recv_sems.at[idx],
                    device_id=(0,), device_id_type=pl.DeviceIdType.MESH,
                )
                recv.wait_recv()

        row = lax.broadcasted_iota(jnp.int32, (SQ, SKV), 0) // BLK
        col = lax.broadcasted_iota(jnp.int32, (SQ, SKV), 1) // BLK
        mask = col <= row
        for b in range(B):
            for h in range(HQ_LOC):
                q_bh = q_all[b][:, h * DH:(h + 1) * DH].astype(jnp.bfloat16)
                k_bh = kv_buf[0, b, h]
                v_bh = kv_buf[1, b, h]
                s = lax.dot_general(
                    q_bh, k_bh, (((1,), (0,)), ((), ())),
                    preferred_element_type=jnp.float32) * 0.125
                s = jnp.where(mask, s, -1e9)
                m = jnp.max(s, axis=1, keepdims=True)
                w = jnp.exp(s - m)
                w = w / jnp.sum(w, axis=1, keepdims=True)
                ctx_buf[b, :, h * DH:(h + 1) * DH] = lax.dot_general(
                    w.astype(jnp.bfloat16), v_bh, (((1,), (1,)), ((), ())),
                    preferred_element_type=jnp.float32)

        for b in range(B):
            acc[b] = jnp.dot(ctx_buf[b], wo_v[...],
                             preferred_element_type=jnp.float32)

        for s_i, bit in enumerate(BFLY_BITS):
            partner = my_pos ^ bit
            bfly_snd[...] = acc[...].astype(jnp.bfloat16)
            rdma = pltpu.make_async_remote_copy(
                src_ref=bfly_snd,
                dst_ref=bfly_buf.at[s_i],
                send_sem=bfly_send_sems.at[s_i],
                recv_sem=bfly_recv_sems.at[s_i],
                device_id=(partner,),
                device_id_type=pl.DeviceIdType.MESH,
            )
            rdma.start()
            rdma.wait()
            acc[...] = acc[...] + bfly_buf[s_i].astype(jnp.float32)

        out_cp = pltpu.make_async_copy(acc, out_ref, local_sems.at[5])
        out_cp.start()
        out_cp.wait()

        @pl.when(my_pos == 0)
        def _():
            for r in scatter_rdmas():
                r.wait_send()

    hbm = pl.BlockSpec(memory_space=pltpu.MemorySpace.HBM)
    return pl.pallas_call(
        body,
        out_shape=jax.ShapeDtypeStruct((B, SQ, D_MODEL), jnp.float32),
        in_specs=[hbm] * 5,
        out_specs=hbm,
        scratch_shapes=[
            pltpu.VMEM((B, SQ, D_MODEL), jnp.float32),
            pltpu.VMEM((D_MODEL, HQ_LOC * DH), jnp.float32),
            pltpu.VMEM((HQ_LOC * DH, D_MODEL), jnp.float32),
            pltpu.VMEM((B, SQ, D_MODEL), jnp.float32),
            pltpu.VMEM((2, B, HQ_LOC, DH, SKV), jnp.bfloat16),
            pltpu.VMEM((2, B, N_DEV * HQ_LOC, DH, SKV), jnp.float32),
            pltpu.VMEM((N_DEV - 1, 2, B, HQ_LOC, DH, SKV), jnp.bfloat16),
            pltpu.VMEM((B, SQ, HQ_LOC * DH), jnp.float32),
            pltpu.VMEM((B, SQ, D_MODEL), jnp.bfloat16),
            pltpu.VMEM((3, B, SQ, D_MODEL), jnp.bfloat16),
            pltpu.SemaphoreType.DMA((N_DEV - 1, 2)),
            pltpu.SemaphoreType.DMA((2,)),
            pltpu.SemaphoreType.DMA((3,)),
            pltpu.SemaphoreType.DMA((3,)),
            pltpu.SemaphoreType.DMA((6,)),
        ],
    )(x, Wq, K_t, V_t, Wo)


# device time: 59213 ns/iter; 1.7266x vs baseline; 1.1057x over previous
import jax
import jax.numpy as jnp
from jax import lax
from jax.experimental import pallas as pl
from jax.experimental.pallas import tpu as pltpu

N_DEV = 8
B = 2
SQ = 256
SKV = 256
HQ_LOC = 4
DH = 64
D_MODEL = 512
BLK = 64
BFLY_BITS = (4, 3, 1)


def kernel(x, Wq, K_ext, V_ext, Wo):
    K_t = jnp.transpose(K_ext, (0, 2, 3, 1))
    V_t = jnp.transpose(V_ext, (0, 2, 3, 1))

    def body(x_ref, wq_ref, k_ref, v_ref, wo_ref, out_ref,
             x_v, wq_v, wo_v, acc, kv_buf, kv_f32, stage_ref, ctx_buf,
             bfly_snd, bfly_buf,
             scat_send_sems, scat_recv_sems, bfly_send_sems, bfly_recv_sems,
             local_sems):
        my_pos = lax.axis_index("i")

        barrier_sem = pltpu.get_barrier_semaphore()
        @pl.when(my_pos != 0)
        def _():
            pl.semaphore_signal(barrier_sem, inc=1, device_id=(0,),
                                device_id_type=pl.DeviceIdType.MESH)

        in_cps = [
            pltpu.make_async_copy(x_ref, x_v, local_sems.at[2]),
            pltpu.make_async_copy(wq_ref, wq_v, local_sems.at[3]),
            pltpu.make_async_copy(wo_ref, wo_v, local_sems.at[4]),
        ]
        for cp in in_cps:
            cp.start()

        def scatter_rdmas():
            rdmas = []
            for p in range(1, N_DEV):
                for idx in range(2):
                    rdmas.append(pltpu.make_async_remote_copy(
                        src_ref=stage_ref.at[p - 1, idx],
                        dst_ref=kv_buf.at[idx],
                        send_sem=scat_send_sems.at[p - 1, idx],
                        recv_sem=scat_recv_sems.at[idx],
                        device_id=(p,),
                        device_id_type=pl.DeviceIdType.MESH,
                    ))
            return rdmas

        @pl.when(my_pos == 0)
        def _():
            cps = [
                pltpu.make_async_copy(k_ref, kv_f32.at[0], local_sems.at[0]),
                pltpu.make_async_copy(v_ref, kv_f32.at[1], local_sems.at[1]),
            ]
            for cp in cps:
                cp.start()
            for cp in cps:
                cp.wait()
            for p in range(1, N_DEV):
                lo = HQ_LOC * p
                stage_ref[p - 1, 0] = kv_f32[0, :, lo:lo + HQ_LOC, :, :].astype(
                    jnp.bfloat16)
                stage_ref[p - 1, 1] = kv_f32[1, :, lo:lo + HQ_LOC, :, :].astype(
                    jnp.bfloat16)
            pl.semaphore_wait(barrier_sem, N_DEV - 1)
            for r in scatter_rdmas():
                r.start()
            kv_buf[0] = kv_f32[0, :, 0:HQ_LOC, :, :].astype(jnp.bfloat16)
            kv_buf[1] = kv_f32[1, :, 0:HQ_LOC, :, :].astype(jnp.bfloat16)

        for cp in in_cps:
            cp.wait()
        q_all = [jnp.dot(x_v[b], wq_v[...],
                         preferred_element_type=jnp.float32) for b in range(B)]

        @pl.when(my_pos != 0)
        def _():
            for idx in range(2):
                recv = pltpu.make_async_remote_copy(
                    src_ref=kv_buf.at[idx], dst_ref=kv_buf.at[idx],
                    send_sem=scat_send_sems.at[0, idx],
                    recv_sem=scat_recv_sems.at[idx],
                    device_id=(0,), device_id_type=pl.DeviceIdType.MESH,
                )
                recv.wait_recv()

        row = lax.broadcasted_iota(jnp.int32, (SQ, SKV), 0) // BLK
        col = lax.broadcasted_iota(jnp.int32, (SQ, SKV), 1) // BLK
        mask = col <= row
        for b in range(B):
            for h in range(HQ_LOC):
                q_bh = q_all[b][:, h * DH:(h + 1) * DH].astype(jnp.bfloat16)
                k_bh = kv_buf[0, b, h]
                v_bh = kv_buf[1, b, h]
                s = lax.dot_general(
                    q_bh, k_bh, (((1,), (0,)), ((), ())),
                    preferred_element_type=jnp.float32) * 0.125
                s = jnp.where(mask, s, -1e9)
                m = jnp.max(s, axis=1, keepdims=True)
                w = jnp.exp(s - m)
                w = w / jnp.sum(w, axis=1, keepdims=True)
                ctx_buf[b, :, h * DH:(h + 1) * DH] = lax.dot_general(
                    w.astype(jnp.bfloat16), v_bh, (((1,), (1,)), ((), ())),
                    preferred_element_type=jnp.float32)

        for b in range(B):
            acc[b] = jnp.dot(ctx_buf[b], wo_v[...],
                             preferred_element_type=jnp.float32)

        for s_i, bit in enumerate(BFLY_BITS):
            partner = my_pos ^ bit
            bfly_snd[...] = acc[...].astype(jnp.bfloat16)
            rdma = pltpu.make_async_remote_copy(
                src_ref=bfly_snd,
                dst_ref=bfly_buf.at[s_i],
                send_sem=bfly_send_sems.at[s_i],
                recv_sem=bfly_recv_sems.at[s_i],
                device_id=(partner,),
                device_id_type=pl.DeviceIdType.MESH,
            )
            rdma.start()
            rdma.wait()
            acc[...] = acc[...] + bfly_buf[s_i].astype(jnp.float32)

        out_cp = pltpu.make_async_copy(acc, out_ref, local_sems.at[5])
        out_cp.start()
        out_cp.wait()

        @pl.when(my_pos == 0)
        def _():
            for r in scatter_rdmas():
                r.wait_send()

    hbm = pl.BlockSpec(memory_space=pltpu.MemorySpace.HBM)
    return pl.pallas_call(
        body,
        out_shape=jax.ShapeDtypeStruct((B, SQ, D_MODEL), jnp.float32),
        in_specs=[hbm] * 5,
        out_specs=hbm,
        scratch_shapes=[
            pltpu.VMEM((B, SQ, D_MODEL), jnp.float32),
            pltpu.VMEM((D_MODEL, HQ_LOC * DH), jnp.float32),
            pltpu.VMEM((HQ_LOC * DH, D_MODEL), jnp.float32),
            pltpu.VMEM((B, SQ, D_MODEL), jnp.float32),
            pltpu.VMEM((2, B, HQ_LOC, DH, SKV), jnp.bfloat16),
            pltpu.VMEM((2, B, N_DEV * HQ_LOC, DH, SKV), jnp.float32),
            pltpu.VMEM((N_DEV - 1, 2, B, HQ_LOC, DH, SKV), jnp.bfloat16),
            pltpu.VMEM((B, SQ, HQ_LOC * DH), jnp.float32),
            pltpu.VMEM((B, SQ, D_MODEL), jnp.bfloat16),
            pltpu.VMEM((3, B, SQ, D_MODEL), jnp.bfloat16),
            pltpu.SemaphoreType.DMA((N_DEV - 1, 2)),
            pltpu.SemaphoreType.DMA((2,)),
            pltpu.SemaphoreType.DMA((3,)),
            pltpu.SemaphoreType.DMA((3,)),
            pltpu.SemaphoreType.DMA((6,)),
        ],
        compiler_params=pltpu.CompilerParams(collective_id=0),
    )(x, Wq, K_t, V_t, Wo)


# device time: 53898 ns/iter; 1.8968x vs baseline; 1.0986x over previous
import jax
import jax.numpy as jnp
from jax import lax
from jax.experimental import pallas as pl
from jax.experimental.pallas import tpu as pltpu

N_DEV = 8
B = 2
SQ = 256
SKV = 256
HQ_LOC = 4
DH = 64
D_MODEL = 512
BLK = 64
BFLY_BITS = (4, 3, 1)


def kernel(x, Wq, K_ext, V_ext, Wo):
    K_t = jnp.transpose(K_ext, (0, 2, 3, 1))
    V_t = jnp.transpose(V_ext, (0, 2, 3, 1))

    def body(x_ref, wq_ref, k_ref, v_ref, wo_ref, out_ref,
             x_v, wq_v, wo_v, acc, kv_buf, kv_f32, stage_ref, ctx_buf,
             bfly_snd, bfly_buf,
             scat_send_sems, scat_recv_sems, bfly_send_sems, bfly_recv_sems,
             local_sems):
        my_pos = lax.axis_index("i")

        barrier_sem = pltpu.get_barrier_semaphore()
        @pl.when(my_pos != 0)
        def _():
            pl.semaphore_signal(barrier_sem, inc=1, device_id=(0,),
                                device_id_type=pl.DeviceIdType.MESH)

        in_cps = [
            pltpu.make_async_copy(x_ref, x_v, local_sems.at[2]),
            pltpu.make_async_copy(wq_ref, wq_v, local_sems.at[3]),
            pltpu.make_async_copy(wo_ref, wo_v, local_sems.at[4]),
        ]
        for cp in in_cps:
            cp.start()

        def scatter_rdmas():
            rdmas = []
            for p in range(1, N_DEV):
                for idx in range(2):
                    rdmas.append(pltpu.make_async_remote_copy(
                        src_ref=stage_ref.at[p - 1, idx],
                        dst_ref=kv_buf.at[idx],
                        send_sem=scat_send_sems.at[p - 1, idx],
                        recv_sem=scat_recv_sems.at[idx],
                        device_id=(p,),
                        device_id_type=pl.DeviceIdType.MESH,
                    ))
            return rdmas

        @pl.when(my_pos == 0)
        def _():
            cps = [
                pltpu.make_async_copy(k_ref, kv_f32.at[0], local_sems.at[0]),
                pltpu.make_async_copy(v_ref, kv_f32.at[1], local_sems.at[1]),
            ]
            for cp in cps:
                cp.start()
            for cp in cps:
                cp.wait()
            for p in range(1, N_DEV):
                lo = HQ_LOC * p
                stage_ref[p - 1, 0] = kv_f32[0, :, lo:lo + HQ_LOC, :, :].astype(
                    jnp.bfloat16)
                stage_ref[p - 1, 1] = kv_f32[1, :, lo:lo + HQ_LOC, :, :].astype(
                    jnp.bfloat16)
            pl.semaphore_wait(barrier_sem, N_DEV - 1)
            for r in scatter_rdmas():
                r.start()
            kv_buf[0] = kv_f32[0, :, 0:HQ_LOC, :, :].astype(jnp.bfloat16)
            kv_buf[1] = kv_f32[1, :, 0:HQ_LOC, :, :].astype(jnp.bfloat16)

        for cp in in_cps:
            cp.wait()
        q_all = [jnp.dot(x_v[b], wq_v[...],
                         preferred_element_type=jnp.float32) for b in range(B)]

        @pl.when(my_pos != 0)
        def _():
            for idx in range(2):
                recv = pltpu.make_async_remote_copy(
                    src_ref=kv_buf.at[idx], dst_ref=kv_buf.at[idx],
                    send_sem=scat_send_sems.at[0, idx],
                    recv_sem=scat_recv_sems.at[idx],
                    device_id=(0,), device_id_type=pl.DeviceIdType.MESH,
                )
                recv.wait_recv()

        row = lax.broadcasted_iota(jnp.int32, (SQ, SKV), 0) // BLK
        col = lax.broadcasted_iota(jnp.int32, (SQ, SKV), 1) // BLK
        mask = col <= row
        for b in range(B):
            for h in range(HQ_LOC):
                q_bh = q_all[b][:, h * DH:(h + 1) * DH].astype(jnp.bfloat16)
                k_bh = kv_buf[0, b, h]
                v_bh = kv_buf[1, b, h]
                s = lax.dot_general(
                    q_bh, k_bh, (((1,), (0,)), ((), ())),
                    preferred_element_type=jnp.float32) * 0.125
                s = jnp.where(mask, s, -1e9)
                m = jnp.max(s, axis=1, keepdims=True)
                w = jnp.exp(s - m)
                w = w / jnp.sum(w, axis=1, keepdims=True)
                ctx_buf[b, :, h * DH:(h + 1) * DH] = lax.dot_general(
                    w.astype(jnp.bfloat16), v_bh, (((1,), (1,)), ((), ())),
                    preferred_element_type=jnp.float32)

        NCH = 2
        CH = SQ // NCH

        def proj_chunk(c):
            for b in range(B):
                acc[b, c * CH:(c + 1) * CH, :] = jnp.dot(
                    ctx_buf[b, c * CH:(c + 1) * CH, :], wo_v[...],
                    preferred_element_type=jnp.float32)

        def bfly_rdma(s_i, c):
            partner = my_pos ^ BFLY_BITS[s_i]
            return pltpu.make_async_remote_copy(
                src_ref=bfly_snd.at[c],
                dst_ref=bfly_buf.at[s_i, c],
                send_sem=bfly_send_sems.at[s_i, c],
                recv_sem=bfly_recv_sems.at[s_i, c],
                device_id=(partner,),
                device_id_type=pl.DeviceIdType.MESH,
            )

        def bfly_send(s_i, c):
            bfly_snd[c] = acc[:, c * CH:(c + 1) * CH, :].astype(jnp.bfloat16)
            r = bfly_rdma(s_i, c)
            r.start()
            return r

        def bfly_finish(r, s_i, c):
            r.wait()
            acc[:, c * CH:(c + 1) * CH, :] = (
                acc[:, c * CH:(c + 1) * CH, :]
                + bfly_buf[s_i, c].astype(jnp.float32))

        inflight = [None, None]
        proj_chunk(0)
        inflight[0] = bfly_send(0, 0)
        proj_chunk(1)
        inflight[1] = bfly_send(0, 1)
        for s_i in range(3):
            for c in range(NCH):
                bfly_finish(inflight[c], s_i, c)
                if s_i < 2:
                    inflight[c] = bfly_send(s_i + 1, c)

        out_cp = pltpu.make_async_copy(acc, out_ref, local_sems.at[5])
        out_cp.start()
        out_cp.wait()

        @pl.when(my_pos == 0)
        def _():
            for r in scatter_rdmas():
                r.wait_send()

    hbm = pl.BlockSpec(memory_space=pltpu.MemorySpace.HBM)
    return pl.pallas_call(
        body,
        out_shape=jax.ShapeDtypeStruct((B, SQ, D_MODEL), jnp.float32),
        in_specs=[hbm] * 5,
        out_specs=hbm,
        scratch_shapes=[
            pltpu.VMEM((B, SQ, D_MODEL), jnp.float32),
            pltpu.VMEM((D_MODEL, HQ_LOC * DH), jnp.float32),
            pltpu.VMEM((HQ_LOC * DH, D_MODEL), jnp.float32),
            pltpu.VMEM((B, SQ, D_MODEL), jnp.float32),
            pltpu.VMEM((2, B, HQ_LOC, DH, SKV), jnp.bfloat16),
            pltpu.VMEM((2, B, N_DEV * HQ_LOC, DH, SKV), jnp.float32),
            pltpu.VMEM((N_DEV - 1, 2, B, HQ_LOC, DH, SKV), jnp.bfloat16),
            pltpu.VMEM((B, SQ, HQ_LOC * DH), jnp.float32),
            pltpu.VMEM((2, B, SQ // 2, D_MODEL), jnp.bfloat16),
            pltpu.VMEM((3, 2, B, SQ // 2, D_MODEL), jnp.bfloat16),
            pltpu.SemaphoreType.DMA((N_DEV - 1, 2)),
            pltpu.SemaphoreType.DMA((2,)),
            pltpu.SemaphoreType.DMA((3, 2)),
            pltpu.SemaphoreType.DMA((3, 2)),
            pltpu.SemaphoreType.DMA((6,)),
        ],
        compiler_params=pltpu.CompilerParams(collective_id=0),
    )(x, Wq, K_t, V_t, Wo)


# device time: 50488 ns/iter; 2.0250x vs baseline; 1.0675x over previous
import jax
import jax.numpy as jnp
from jax import lax
from jax.experimental import pallas as pl
from jax.experimental.pallas import tpu as pltpu

N_DEV = 8
B = 2
SQ = 256
SKV = 256
HQ_LOC = 4
DH = 64
D_MODEL = 512
BLK = 64
BFLY_BITS = (4, 3, 1)


def kernel(x, Wq, K_ext, V_ext, Wo):
    K_t = jnp.transpose(K_ext, (0, 2, 3, 1))
    V_t = jnp.transpose(V_ext, (0, 2, 3, 1))

    def body(x_ref, wq_ref, k_ref, v_ref, wo_ref, out_ref,
             x_v, wq_v, wo_v, acc, kv_buf, kv_f32, stage_ref, relay_buf,
             ctx_buf, bfly_snd, bfly_buf,
             scat_send_sems, scat_recv_sems, bfly_send_sems, bfly_recv_sems,
             local_sems, relay_recv_sems, fwd_send_sems):
        my_pos = lax.axis_index("i")

        barrier_sem = pltpu.get_barrier_semaphore()
        @pl.when(my_pos != 0)
        def _():
            pl.semaphore_signal(barrier_sem, inc=1, device_id=(0,),
                                device_id_type=pl.DeviceIdType.MESH)

        in_cps = [
            pltpu.make_async_copy(x_ref, x_v, local_sems.at[2]),
            pltpu.make_async_copy(wq_ref, wq_v, local_sems.at[3]),
            pltpu.make_async_copy(wo_ref, wo_v, local_sems.at[4]),
        ]
        for cp in in_cps:
            cp.start()

        def scatter_rdmas():
            rdmas = []
            for idx in range(2):
                rdmas.append(pltpu.make_async_remote_copy(
                    src_ref=stage_ref.at[5, idx],
                    dst_ref=relay_buf.at[idx],
                    send_sem=scat_send_sems.at[5, idx],
                    recv_sem=relay_recv_sems.at[idx],
                    device_id=(4,),
                    device_id_type=pl.DeviceIdType.MESH,
                ))
            for p in (1, 2, 3, 4, 5, 7):
                for idx in range(2):
                    rdmas.append(pltpu.make_async_remote_copy(
                        src_ref=stage_ref.at[p - 1, idx],
                        dst_ref=kv_buf.at[idx],
                        send_sem=scat_send_sems.at[p - 1, idx],
                        recv_sem=scat_recv_sems.at[idx],
                        device_id=(p,),
                        device_id_type=pl.DeviceIdType.MESH,
                    ))
            return rdmas

        def fwd_rdmas():
            return [pltpu.make_async_remote_copy(
                src_ref=relay_buf.at[idx],
                dst_ref=kv_buf.at[idx],
                send_sem=fwd_send_sems.at[idx],
                recv_sem=scat_recv_sems.at[idx],
                device_id=(6,),
                device_id_type=pl.DeviceIdType.MESH,
            ) for idx in range(2)]

        @pl.when(my_pos == 0)
        def _():
            cps = [
                pltpu.make_async_copy(k_ref, kv_f32.at[0], local_sems.at[0]),
                pltpu.make_async_copy(v_ref, kv_f32.at[1], local_sems.at[1]),
            ]
            for cp in cps:
                cp.start()
            for cp in cps:
                cp.wait()
            for p in range(1, N_DEV):
                lo = HQ_LOC * p
                stage_ref[p - 1, 0] = kv_f32[0, :, lo:lo + HQ_LOC, :, :].astype(
                    jnp.bfloat16)
                stage_ref[p - 1, 1] = kv_f32[1, :, lo:lo + HQ_LOC, :, :].astype(
                    jnp.bfloat16)
            pl.semaphore_wait(barrier_sem, N_DEV - 1)
            for r in scatter_rdmas():
                r.start()
            kv_buf[0] = kv_f32[0, :, 0:HQ_LOC, :, :].astype(jnp.bfloat16)
            kv_buf[1] = kv_f32[1, :, 0:HQ_LOC, :, :].astype(jnp.bfloat16)

        for cp in in_cps:
            cp.wait()
        q_all = [jnp.dot(x_v[b], wq_v[...],
                         preferred_element_type=jnp.float32) for b in range(B)]

        @pl.when(my_pos == 4)
        def _():
            for idx in range(2):
                rr = pltpu.make_async_remote_copy(
                    src_ref=relay_buf.at[idx], dst_ref=relay_buf.at[idx],
                    send_sem=fwd_send_sems.at[idx],
                    recv_sem=relay_recv_sems.at[idx],
                    device_id=(0,), device_id_type=pl.DeviceIdType.MESH,
                )
                rr.wait_recv()
            for f in fwd_rdmas():
                f.start()

        @pl.when(my_pos != 0)
        def _():
            for idx in range(2):
                recv = pltpu.make_async_remote_copy(
                    src_ref=kv_buf.at[idx], dst_ref=kv_buf.at[idx],
                    send_sem=scat_send_sems.at[0, idx],
                    recv_sem=scat_recv_sems.at[idx],
                    device_id=(0,), device_id_type=pl.DeviceIdType.MESH,
                )
                recv.wait_recv()

        row = lax.broadcasted_iota(jnp.int32, (SQ, SKV), 0) // BLK
        col = lax.broadcasted_iota(jnp.int32, (SQ, SKV), 1) // BLK
        mask = col <= row
        for b in range(B):
            for h in range(HQ_LOC):
                q_bh = q_all[b][:, h * DH:(h + 1) * DH].astype(jnp.bfloat16)
                k_bh = kv_buf[0, b, h]
                v_bh = kv_buf[1, b, h]
                s = lax.dot_general(
                    q_bh, k_bh, (((1,), (0,)), ((), ())),
                    preferred_element_type=jnp.float32) * 0.125
                s = jnp.where(mask, s, -1e9)
                m = jnp.max(s, axis=1, keepdims=True)
                w = jnp.exp(s - m)
                w = w / jnp.sum(w, axis=1, keepdims=True)
                ctx_buf[b, :, h * DH:(h + 1) * DH] = lax.dot_general(
                    w.astype(jnp.bfloat16), v_bh, (((1,), (1,)), ((), ())),
                    preferred_element_type=jnp.float32)

        NCH = 2
        CH = SQ // NCH

        def proj_chunk(c):
            for b in range(B):
                acc[b, c * CH:(c + 1) * CH, :] = jnp.dot(
                    ctx_buf[b, c * CH:(c + 1) * CH, :], wo_v[...],
                    preferred_element_type=jnp.float32)

        def bfly_rdma(s_i, c):
            partner = my_pos ^ BFLY_BITS[s_i]
            return pltpu.make_async_remote_copy(
                src_ref=bfly_snd.at[c],
                dst_ref=bfly_buf.at[s_i, c],
                send_sem=bfly_send_sems.at[s_i, c],
                recv_sem=bfly_recv_sems.at[s_i, c],
                device_id=(partner,),
                device_id_type=pl.DeviceIdType.MESH,
            )

        def bfly_send(s_i, c):
            bfly_snd[c] = acc[:, c * CH:(c + 1) * CH, :].astype(jnp.bfloat16)
            r = bfly_rdma(s_i, c)
            r.start()
            return r

        def bfly_finish(r, s_i, c):
            r.wait()
            acc[:, c * CH:(c + 1) * CH, :] = (
                acc[:, c * CH:(c + 1) * CH, :]
                + bfly_buf[s_i, c].astype(jnp.float32))

        inflight = [None, None]
        proj_chunk(0)
        inflight[0] = bfly_send(0, 0)
        proj_chunk(1)
        inflight[1] = bfly_send(0, 1)
        for s_i in range(3):
            for c in range(NCH):
                bfly_finish(inflight[c], s_i, c)
                if s_i < 2:
                    inflight[c] = bfly_send(s_i + 1, c)

        out_cp = pltpu.make_async_copy(acc, out_ref, local_sems.at[5])
        out_cp.start()
        out_cp.wait()

        @pl.when(my_pos == 0)
        def _():
            for r in scatter_rdmas():
                r.wait_send()

        @pl.when(my_pos == 4)
        def _():
            for f in fwd_rdmas():
                f.wait_send()

    hbm = pl.BlockSpec(memory_space=pltpu.MemorySpace.HBM)
    return pl.pallas_call(
        body,
        out_shape=jax.ShapeDtypeStruct((B, SQ, D_MODEL), jnp.float32),
        in_specs=[hbm] * 5,
        out_specs=hbm,
        scratch_shapes=[
            pltpu.VMEM((B, SQ, D_MODEL), jnp.float32),
            pltpu.VMEM((D_MODEL, HQ_LOC * DH), jnp.float32),
            pltpu.VMEM((HQ_LOC * DH, D_MODEL), jnp.float32),
            pltpu.VMEM((B, SQ, D_MODEL), jnp.float32),
            pltpu.VMEM((2, B, HQ_LOC, DH, SKV), jnp.bfloat16),
            pltpu.VMEM((2, B, N_DEV * HQ_LOC, DH, SKV), jnp.float32),
            pltpu.VMEM((N_DEV - 1, 2, B, HQ_LOC, DH, SKV), jnp.bfloat16),
            pltpu.VMEM((2, B, HQ_LOC, DH, SKV), jnp.bfloat16),
            pltpu.VMEM((B, SQ, HQ_LOC * DH), jnp.float32),
            pltpu.VMEM((2, B, SQ // 2, D_MODEL), jnp.bfloat16),
            pltpu.VMEM((3, 2, B, SQ // 2, D_MODEL), jnp.bfloat16),
            pltpu.SemaphoreType.DMA((N_DEV - 1, 2)),
            pltpu.SemaphoreType.DMA((2,)),
            pltpu.SemaphoreType.DMA((3, 2)),
            pltpu.SemaphoreType.DMA((3, 2)),
            pltpu.SemaphoreType.DMA((6,)),
            pltpu.SemaphoreType.DMA((2,)),
            pltpu.SemaphoreType.DMA((2,)),
        ],
        compiler_params=pltpu.CompilerParams(collective_id=0),
    )(x, Wq, K_t, V_t, Wo)


# device time: 49056 ns/iter; 2.0841x vs baseline; 1.0292x over previous
import jax
import jax.numpy as jnp
from jax import lax
from jax.experimental import pallas as pl
from jax.experimental.pallas import tpu as pltpu

N_DEV = 8
B = 2
SQ = 256
SKV = 256
HQ_LOC = 4
DH = 64
D_MODEL = 512
BLK = 64
BFLY_BITS = (4, 3, 1)


def kernel(x, Wq, K_ext, V_ext, Wo):
    K_t = jnp.transpose(K_ext, (0, 2, 3, 1))
    V_t = jnp.transpose(V_ext, (0, 2, 3, 1))

    def body(x_ref, wq_ref, k_ref, v_ref, wo_ref, out_ref,
             x_v, wq_v, wo_v, acc, kv_buf, kv_f32, stage_ref, relay_buf,
             w_buf, ctx_buf, bfly_snd, bfly_buf,
             scat_send_sems, scat_recv_sems, bfly_send_sems, bfly_recv_sems,
             local_sems, relay_recv_sems, fwd_send_sems):
        my_pos = lax.axis_index("i")

        barrier_sem = pltpu.get_barrier_semaphore()
        @pl.when(my_pos != 0)
        def _():
            pl.semaphore_signal(barrier_sem, inc=1, device_id=(0,),
                                device_id_type=pl.DeviceIdType.MESH)

        in_cps = [
            pltpu.make_async_copy(x_ref, x_v, local_sems.at[2]),
            pltpu.make_async_copy(wq_ref, wq_v, local_sems.at[3]),
            pltpu.make_async_copy(wo_ref, wo_v, local_sems.at[4]),
        ]
        for cp in in_cps:
            cp.start()

        def scatter_rdmas(idx):
            rdmas = [pltpu.make_async_remote_copy(
                src_ref=stage_ref.at[5, idx],
                dst_ref=relay_buf.at[idx],
                send_sem=scat_send_sems.at[5, idx],
                recv_sem=relay_recv_sems.at[idx],
                device_id=(4,),
                device_id_type=pl.DeviceIdType.MESH,
            )]
            for p in (1, 2, 3, 4, 5, 7):
                rdmas.append(pltpu.make_async_remote_copy(
                    src_ref=stage_ref.at[p - 1, idx],
                    dst_ref=kv_buf.at[idx],
                    send_sem=scat_send_sems.at[p - 1, idx],
                    recv_sem=scat_recv_sems.at[idx],
                    device_id=(p,),
                    device_id_type=pl.DeviceIdType.MESH,
                ))
            return rdmas

        def fwd_rdma(idx):
            return pltpu.make_async_remote_copy(
                src_ref=relay_buf.at[idx],
                dst_ref=kv_buf.at[idx],
                send_sem=fwd_send_sems.at[idx],
                recv_sem=scat_recv_sems.at[idx],
                device_id=(6,),
                device_id_type=pl.DeviceIdType.MESH,
            )

        def relay_wait_rdma(idx):
            return pltpu.make_async_remote_copy(
                src_ref=relay_buf.at[idx], dst_ref=relay_buf.at[idx],
                send_sem=fwd_send_sems.at[idx],
                recv_sem=relay_recv_sems.at[idx],
                device_id=(0,), device_id_type=pl.DeviceIdType.MESH,
            )

        def kv_wait_rdma(idx):
            return pltpu.make_async_remote_copy(
                src_ref=kv_buf.at[idx], dst_ref=kv_buf.at[idx],
                send_sem=scat_send_sems.at[0, idx],
                recv_sem=scat_recv_sems.at[idx],
                device_id=(0,), device_id_type=pl.DeviceIdType.MESH,
            )

        @pl.when(my_pos == 0)
        def _():
            cps = [
                pltpu.make_async_copy(k_ref, kv_f32.at[0], local_sems.at[0]),
                pltpu.make_async_copy(v_ref, kv_f32.at[1], local_sems.at[1]),
            ]
            for cp in cps:
                cp.start()
            for cp in cps:
                cp.wait()
            for p in range(1, N_DEV):
                lo = HQ_LOC * p
                stage_ref[p - 1, 0] = kv_f32[0, :, lo:lo + HQ_LOC, :, :].astype(
                    jnp.bfloat16)
            pl.semaphore_wait(barrier_sem, N_DEV - 1)
            for r in scatter_rdmas(0):
                r.start()
            for p in range(1, N_DEV):
                lo = HQ_LOC * p
                stage_ref[p - 1, 1] = kv_f32[1, :, lo:lo + HQ_LOC, :, :].astype(
                    jnp.bfloat16)
            for r in scatter_rdmas(1):
                r.start()
            kv_buf[0] = kv_f32[0, :, 0:HQ_LOC, :, :].astype(jnp.bfloat16)
            kv_buf[1] = kv_f32[1, :, 0:HQ_LOC, :, :].astype(jnp.bfloat16)

        for cp in in_cps:
            cp.wait()
        q_all = [jnp.dot(x_v[b], wq_v[...],
                         preferred_element_type=jnp.float32) for b in range(B)]

        @pl.when(my_pos == 4)
        def _():
            relay_wait_rdma(0).wait_recv()
            fwd_rdma(0).start()

        @pl.when(my_pos != 0)
        def _():
            kv_wait_rdma(0).wait_recv()

        row = lax.broadcasted_iota(jnp.int32, (SQ, SKV), 0) // BLK
        col = lax.broadcasted_iota(jnp.int32, (SQ, SKV), 1) // BLK
        mask = col <= row
        for b in range(B):
            for h in range(HQ_LOC):
                q_bh = q_all[b][:, h * DH:(h + 1) * DH].astype(jnp.bfloat16)
                k_bh = kv_buf[0, b, h]
                s = lax.dot_general(
                    q_bh, k_bh, (((1,), (0,)), ((), ())),
                    preferred_element_type=jnp.float32) * 0.125
                s = jnp.where(mask, s, -1e9)
                m = jnp.max(s, axis=1, keepdims=True)
                w = jnp.exp(s - m)
                w_buf[b, h] = (w / jnp.sum(w, axis=1, keepdims=True)).astype(
                    jnp.bfloat16)

        @pl.when(my_pos == 4)
        def _():
            relay_wait_rdma(1).wait_recv()
            fwd_rdma(1).start()

        @pl.when(my_pos != 0)
        def _():
            kv_wait_rdma(1).wait_recv()

        for b in range(B):
            for h in range(HQ_LOC):
                ctx_buf[b, :, h * DH:(h + 1) * DH] = lax.dot_general(
                    w_buf[b, h], kv_buf[1, b, h], (((1,), (1,)), ((), ())),
                    preferred_element_type=jnp.float32)

        NCH = 2
        CH = SQ // NCH

        def proj_chunk(c):
            for b in range(B):
                acc[b, c * CH:(c + 1) * CH, :] = jnp.dot(
                    ctx_buf[b, c * CH:(c + 1) * CH, :], wo_v[...],
                    preferred_element_type=jnp.float32)

        def bfly_rdma(s_i, c):
            partner = my_pos ^ BFLY_BITS[s_i]
            return pltpu.make_async_remote_copy(
                src_ref=bfly_snd.at[c],
                dst_ref=bfly_buf.at[s_i, c],
                send_sem=bfly_send_sems.at[s_i, c],
                recv_sem=bfly_recv_sems.at[s_i, c],
                device_id=(partner,),
                device_id_type=pl.DeviceIdType.MESH,
            )

        def bfly_send(s_i, c):
            bfly_snd[c] = acc[:, c * CH:(c + 1) * CH, :].astype(jnp.bfloat16)
            r = bfly_rdma(s_i, c)
            r.start()
            return r

        def bfly_finish(r, s_i, c):
            r.wait()
            acc[:, c * CH:(c + 1) * CH, :] = (
                acc[:, c * CH:(c + 1) * CH, :]
                + bfly_buf[s_i, c].astype(jnp.float32))

        inflight = [None, None]
        proj_chunk(0)
        inflight[0] = bfly_send(0, 0)
        proj_chunk(1)
        inflight[1] = bfly_send(0, 1)
        for s_i in range(3):
            for c in range(NCH):
                bfly_finish(inflight[c], s_i, c)
                if s_i < 2:
                    inflight[c] = bfly_send(s_i + 1, c)

        out_cp = pltpu.make_async_copy(acc, out_ref, local_sems.at[5])
        out_cp.start()
        out_cp.wait()

        @pl.when(my_pos == 0)
        def _():
            for idx in range(2):
                for r in scatter_rdmas(idx):
                    r.wait_send()

        @pl.when(my_pos == 4)
        def _():
            for idx in range(2):
                fwd_rdma(idx).wait_send()

    hbm = pl.BlockSpec(memory_space=pltpu.MemorySpace.HBM)
    return pl.pallas_call(
        body,
        out_shape=jax.ShapeDtypeStruct((B, SQ, D_MODEL), jnp.float32),
        in_specs=[hbm] * 5,
        out_specs=hbm,
        scratch_shapes=[
            pltpu.VMEM((B, SQ, D_MODEL), jnp.float32),
            pltpu.VMEM((D_MODEL, HQ_LOC * DH), jnp.float32),
            pltpu.VMEM((HQ_LOC * DH, D_MODEL), jnp.float32),
            pltpu.VMEM((B, SQ, D_MODEL), jnp.float32),
            pltpu.VMEM((2, B, HQ_LOC, DH, SKV), jnp.bfloat16),
            pltpu.VMEM((2, B, N_DEV * HQ_LOC, DH, SKV), jnp.float32),
            pltpu.VMEM((N_DEV - 1, 2, B, HQ_LOC, DH, SKV), jnp.bfloat16),
            pltpu.VMEM((2, B, HQ_LOC, DH, SKV), jnp.bfloat16),
            pltpu.VMEM((B, HQ_LOC, SQ, SKV), jnp.bfloat16),
            pltpu.VMEM((B, SQ, HQ_LOC * DH), jnp.float32),
            pltpu.VMEM((2, B, SQ // 2, D_MODEL), jnp.bfloat16),
            pltpu.VMEM((3, 2, B, SQ // 2, D_MODEL), jnp.bfloat16),
            pltpu.SemaphoreType.DMA((N_DEV - 1, 2)),
            pltpu.SemaphoreType.DMA((2,)),
            pltpu.SemaphoreType.DMA((3, 2)),
            pltpu.SemaphoreType.DMA((3, 2)),
            pltpu.SemaphoreType.DMA((6,)),
            pltpu.SemaphoreType.DMA((2,)),
            pltpu.SemaphoreType.DMA((2,)),
        ],
        compiler_params=pltpu.CompilerParams(collective_id=0),
    )(x, Wq, K_t, V_t, Wo)


# device time: 46419 ns/iter; 2.2025x vs baseline; 1.0568x over previous
import jax
import jax.numpy as jnp
from jax import lax
from jax.experimental import pallas as pl
from jax.experimental.pallas import tpu as pltpu

N_DEV = 8
B = 2
SQ = 256
SKV = 256
HALF = SKV // 2
HQ_LOC = 4
DH = 64
D_MODEL = 512
BLK = 64
BFLY_BITS = (4, 3, 1)
SLOTS = ((0, 0), (1, 0), (0, 1), (1, 1))


def kernel(x, Wq, K_ext, V_ext, Wo):
    K_t = jnp.transpose(K_ext, (0, 2, 3, 1))
    V_t = jnp.transpose(V_ext, (0, 2, 3, 1))

    def body(x_ref, wq_ref, k_ref, v_ref, wo_ref, out_ref,
             x_v, wq_v, wo_v, acc, kv_buf, kv_f32, stage_ref, relay_buf,
             w_buf, ctx_buf, bfly_snd, bfly_buf,
             scat_send_sems, scat_recv_sems, bfly_send_sems, bfly_recv_sems,
             local_sems, relay_recv_sems, fwd_send_sems):
        my_pos = lax.axis_index("i")

        barrier_sem = pltpu.get_barrier_semaphore()
        @pl.when(my_pos != 0)
        def _():
            pl.semaphore_signal(barrier_sem, inc=1, device_id=(0,),
                                device_id_type=pl.DeviceIdType.MESH)

        in_cps = [
            pltpu.make_async_copy(x_ref, x_v, local_sems.at[2]),
            pltpu.make_async_copy(wq_ref, wq_v, local_sems.at[3]),
            pltpu.make_async_copy(wo_ref, wo_v, local_sems.at[4]),
        ]
        for cp in in_cps:
            cp.start()

        def slot_id(kv, half):
            return kv * 2 + half

        def scatter_rdmas(s):
            rdmas = [pltpu.make_async_remote_copy(
                src_ref=stage_ref.at[5, s],
                dst_ref=relay_buf.at[s],
                send_sem=scat_send_sems.at[5, s],
                recv_sem=relay_recv_sems.at[s],
                device_id=(4,),
                device_id_type=pl.DeviceIdType.MESH,
            )]
            for p in (1, 2, 3, 4, 5, 7):
                rdmas.append(pltpu.make_async_remote_copy(
                    src_ref=stage_ref.at[p - 1, s],
                    dst_ref=kv_buf.at[s],
                    send_sem=scat_send_sems.at[p - 1, s],
                    recv_sem=scat_recv_sems.at[s],
                    device_id=(p,),
                    device_id_type=pl.DeviceIdType.MESH,
                ))
            return rdmas

        def fwd_rdma(s):
            return pltpu.make_async_remote_copy(
                src_ref=relay_buf.at[s],
                dst_ref=kv_buf.at[s],
                send_sem=fwd_send_sems.at[s],
                recv_sem=scat_recv_sems.at[s],
                device_id=(6,),
                device_id_type=pl.DeviceIdType.MESH,
            )

        def relay_wait(s):
            pltpu.make_async_remote_copy(
                src_ref=relay_buf.at[s], dst_ref=relay_buf.at[s],
                send_sem=fwd_send_sems.at[s],
                recv_sem=relay_recv_sems.at[s],
                device_id=(0,), device_id_type=pl.DeviceIdType.MESH,
            ).wait_recv()

        def kv_wait(s):
            pltpu.make_async_remote_copy(
                src_ref=kv_buf.at[s], dst_ref=kv_buf.at[s],
                send_sem=scat_send_sems.at[0, s],
                recv_sem=scat_recv_sems.at[s],
                device_id=(0,), device_id_type=pl.DeviceIdType.MESH,
            ).wait_recv()

        @pl.when(my_pos == 0)
        def _():
            cps = [
                pltpu.make_async_copy(k_ref, kv_f32.at[0], local_sems.at[0]),
                pltpu.make_async_copy(v_ref, kv_f32.at[1], local_sems.at[1]),
            ]
            for cp in cps:
                cp.start()
            cps[0].wait()
            for kv, half in SLOTS:
                s = slot_id(kv, half)
                if kv == 1 and half == 0:
                    cps[1].wait()
                lo_c = half * HALF
                for p in range(1, N_DEV):
                    lo = HQ_LOC * p
                    stage_ref[p - 1, s] = kv_f32[
                        kv, :, lo:lo + HQ_LOC, :, lo_c:lo_c + HALF].astype(
                        jnp.bfloat16)
                if kv == 0 and half == 0:
                    pl.semaphore_wait(barrier_sem, N_DEV - 1)
                for r in scatter_rdmas(s):
                    r.start()
            for kv, half in SLOTS:
                s = slot_id(kv, half)
                lo_c = half * HALF
                kv_buf[s] = kv_f32[kv, :, 0:HQ_LOC, :, lo_c:lo_c + HALF].astype(
                    jnp.bfloat16)

        for cp in in_cps:
            cp.wait()
        q_all = [jnp.dot(x_v[b], wq_v[...],
                         preferred_element_type=jnp.float32) for b in range(B)]

        row = lax.broadcasted_iota(jnp.int32, (SQ, SKV), 0) // BLK
        col = lax.broadcasted_iota(jnp.int32, (SQ, SKV), 1) // BLK
        mask = col <= row

        def q_slice(b, h, c):
            return q_all[b][c * HALF:(c + 1) * HALF,
                            h * DH:(h + 1) * DH].astype(jnp.bfloat16)

        def softmax_rows(s, msk):
            s = jnp.where(msk, s, -1e9)
            m = jnp.max(s, axis=1, keepdims=True)
            w = jnp.exp(s - m)
            return (w / jnp.sum(w, axis=1, keepdims=True)).astype(jnp.bfloat16)

        @pl.when(my_pos == 4)
        def _():
            relay_wait(slot_id(0, 0))
            fwd_rdma(slot_id(0, 0)).start()

        @pl.when(my_pos != 0)
        def _():
            kv_wait(slot_id(0, 0))

        for b in range(B):
            for h in range(HQ_LOC):
                s = lax.dot_general(
                    q_slice(b, h, 0), kv_buf[slot_id(0, 0), b, h],
                    (((1,), (0,)), ((), ())),
                    preferred_element_type=jnp.float32) * 0.125
                w_buf[b, h, 0:HALF, 0:HALF] = softmax_rows(
                    s, mask[0:HALF, 0:HALF])

        @pl.when(my_pos == 4)
        def _():
            relay_wait(slot_id(1, 0))
            fwd_rdma(slot_id(1, 0)).start()

        @pl.when(my_pos != 0)
        def _():
            kv_wait(slot_id(1, 0))

        for b in range(B):
            for h in range(HQ_LOC):
                ctx_buf[b, 0:HALF, h * DH:(h + 1) * DH] = lax.dot_general(
                    w_buf[b, h, 0:HALF, 0:HALF], kv_buf[slot_id(1, 0), b, h],
                    (((1,), (1,)), ((), ())),
                    preferred_element_type=jnp.float32)

        def proj_chunk(c):
            for b in range(B):
                acc[b, c * HALF:(c + 1) * HALF, :] = jnp.dot(
                    ctx_buf[b, c * HALF:(c + 1) * HALF, :], wo_v[...],
                    preferred_element_type=jnp.float32)

        def bfly_send(s_i, c):
            partner = my_pos ^ BFLY_BITS[s_i]
            bfly_snd[c] = acc[:, c * HALF:(c + 1) * HALF, :].astype(jnp.bfloat16)
            r = pltpu.make_async_remote_copy(
                src_ref=bfly_snd.at[c],
                dst_ref=bfly_buf.at[s_i, c],
                send_sem=bfly_send_sems.at[s_i, c],
                recv_sem=bfly_recv_sems.at[s_i, c],
                device_id=(partner,),
                device_id_type=pl.DeviceIdType.MESH,
            )
            r.start()
            return r

        def bfly_finish(r, s_i, c):
            r.wait()
            acc[:, c * HALF:(c + 1) * HALF, :] = (
                acc[:, c * HALF:(c + 1) * HALF, :]
                + bfly_buf[s_i, c].astype(jnp.float32))

        inflight = [None, None]
        proj_chunk(0)
        inflight[0] = bfly_send(0, 0)

        @pl.when(my_pos == 4)
        def _():
            relay_wait(slot_id(0, 1))
            fwd_rdma(slot_id(0, 1)).start()

        @pl.when(my_pos != 0)
        def _():
            kv_wait(slot_id(0, 1))

        for b in range(B):
            for h in range(HQ_LOC):
                sa = lax.dot_general(
                    q_slice(b, h, 1), kv_buf[slot_id(0, 0), b, h],
                    (((1,), (0,)), ((), ())),
                    preferred_element_type=jnp.float32)
                sb = lax.dot_general(
                    q_slice(b, h, 1), kv_buf[slot_id(0, 1), b, h],
                    (((1,), (0,)), ((), ())),
                    preferred_element_type=jnp.float32)
                s = jnp.concatenate([sa, sb], axis=1) * 0.125
                w_buf[b, h, HALF:SQ, :] = softmax_rows(s, mask[HALF:SQ, :])

        @pl.when(my_pos == 4)
        def _():
            relay_wait(slot_id(1, 1))
            fwd_rdma(slot_id(1, 1)).start()

        @pl.when(my_pos != 0)
        def _():
            kv_wait(slot_id(1, 1))

        for b in range(B):
            for h in range(HQ_LOC):
                ctx_buf[b, HALF:SQ, h * DH:(h + 1) * DH] = (
                    lax.dot_general(
                        w_buf[b, h, HALF:SQ, 0:HALF],
                        kv_buf[slot_id(1, 0), b, h],
                        (((1,), (1,)), ((), ())),
                        preferred_element_type=jnp.float32)
                    + lax.dot_general(
                        w_buf[b, h, HALF:SQ, HALF:SKV],
                        kv_buf[slot_id(1, 1), b, h],
                        (((1,), (1,)), ((), ())),
                        preferred_element_type=jnp.float32))

        proj_chunk(1)
        inflight[1] = bfly_send(0, 1)
        for s_i in range(3):
            for c in range(2):
                bfly_finish(inflight[c], s_i, c)
                if s_i < 2:
                    inflight[c] = bfly_send(s_i + 1, c)

        out_cp = pltpu.make_async_copy(acc, out_ref, local_sems.at[5])
        out_cp.start()
        out_cp.wait()

        @pl.when(my_pos == 0)
        def _():
            for kv, half in SLOTS:
                for r in scatter_rdmas(slot_id(kv, half)):
                    r.wait_send()

        @pl.when(my_pos == 4)
        def _():
            for s in range(4):
                fwd_rdma(s).wait_send()

    hbm = pl.BlockSpec(memory_space=pltpu.MemorySpace.HBM)
    return pl.pallas_call(
        body,
        out_shape=jax.ShapeDtypeStruct((B, SQ, D_MODEL), jnp.float32),
        in_specs=[hbm] * 5,
        out_specs=hbm,
        scratch_shapes=[
            pltpu.VMEM((B, SQ, D_MODEL), jnp.float32),
            pltpu.VMEM((D_MODEL, HQ_LOC * DH), jnp.float32),
            pltpu.VMEM((HQ_LOC * DH, D_MODEL), jnp.float32),
            pltpu.VMEM((B, SQ, D_MODEL), jnp.float32),
            pltpu.VMEM((4, B, HQ_LOC, DH, HALF), jnp.bfloat16),
            pltpu.VMEM((2, B, N_DEV * HQ_LOC, DH, SKV), jnp.float32),
            pltpu.VMEM((N_DEV - 1, 4, B, HQ_LOC, DH, HALF), jnp.bfloat16),
            pltpu.VMEM((4, B, HQ_LOC, DH, HALF), jnp.bfloat16),
            pltpu.VMEM((B, HQ_LOC, SQ, SKV), jnp.bfloat16),
            pltpu.VMEM((B, SQ, HQ_LOC * DH), jnp.float32),
            pltpu.VMEM((2, B, SQ // 2, D_MODEL), jnp.bfloat16),
            pltpu.VMEM((3, 2, B, SQ // 2, D_MODEL), jnp.bfloat16),
            pltpu.SemaphoreType.DMA((N_DEV - 1, 4)),
            pltpu.SemaphoreType.DMA((4,)),
            pltpu.SemaphoreType.DMA((3, 2)),
            pltpu.SemaphoreType.DMA((3, 2)),
            pltpu.SemaphoreType.DMA((6,)),
            pltpu.SemaphoreType.DMA((4,)),
            pltpu.SemaphoreType.DMA((4,)),
        ],
        compiler_params=pltpu.CompilerParams(collective_id=0),
    )(x, Wq, K_t, V_t, Wo)


# device time: 45047 ns/iter; 2.2695x vs baseline; 1.0305x over previous
import jax
import jax.numpy as jnp
from jax import lax
from jax.experimental import pallas as pl
from jax.experimental.pallas import tpu as pltpu

N_DEV = 8
B = 2
SQ = 256
SKV = 256
HALF = SKV // 2
HQ_LOC = 4
DH = 64
D_MODEL = 512
BLK = 64
BFLY_BITS = (4, 3, 1)
SLOTS = ((0, 0), (1, 0), (0, 1), (1, 1))


def kernel(x, Wq, K_ext, V_ext, Wo):
    K_t = jnp.transpose(K_ext, (0, 2, 3, 1))
    V_t = jnp.transpose(V_ext, (0, 2, 3, 1))

    def body(x_ref, wq_ref, k_ref, v_ref, wo_ref, out_ref,
             x_v, wq_v, wo_v, acc, kv_buf, kv_f32, stage_ref, relay_buf,
             w_buf, ctx_buf, bfly_snd, bfly_buf,
             scat_send_sems, scat_recv_sems, bfly_send_sems, bfly_recv_sems,
             local_sems, relay_recv_sems, fwd_send_sems):
        my_pos = lax.axis_index("i")

        barrier_sem = pltpu.get_barrier_semaphore()
        @pl.when(my_pos != 0)
        def _():
            pl.semaphore_signal(barrier_sem, inc=1, device_id=(0,),
                                device_id_type=pl.DeviceIdType.MESH)

        in_cps = [
            pltpu.make_async_copy(x_ref, x_v, local_sems.at[2]),
            pltpu.make_async_copy(wq_ref, wq_v, local_sems.at[3]),
            pltpu.make_async_copy(wo_ref, wo_v, local_sems.at[4]),
        ]
        for cp in in_cps:
            cp.start()

        def slot_id(kv, half):
            return kv * 2 + half

        RELAY_IDX = {(6, 0): 0, (6, 2): 1, (6, 1): 2, (6, 3): 3,
                     (5, 1): 4, (5, 3): 5}

        def scatter_rdmas(s):
            rdmas = []
            for p in (6, 5):
                if (p, s) in RELAY_IDX:
                    rdmas.append(pltpu.make_async_remote_copy(
                        src_ref=stage_ref.at[p - 1, s],
                        dst_ref=relay_buf.at[RELAY_IDX[(p, s)]],
                        send_sem=scat_send_sems.at[p - 1, s],
                        recv_sem=relay_recv_sems.at[RELAY_IDX[(p, s)]],
                        device_id=(4,),
                        device_id_type=pl.DeviceIdType.MESH,
                    ))
            for p in (1, 2, 3, 4, 5, 7):
                if (p, s) in RELAY_IDX:
                    continue
                rdmas.append(pltpu.make_async_remote_copy(
                    src_ref=stage_ref.at[p - 1, s],
                    dst_ref=kv_buf.at[s],
                    send_sem=scat_send_sems.at[p - 1, s],
                    recv_sem=scat_recv_sems.at[s],
                    device_id=(p,),
                    device_id_type=pl.DeviceIdType.MESH,
                ))
            return rdmas

        def fwd_rdma(p, s):
            return pltpu.make_async_remote_copy(
                src_ref=relay_buf.at[RELAY_IDX[(p, s)]],
                dst_ref=kv_buf.at[s],
                send_sem=fwd_send_sems.at[RELAY_IDX[(p, s)]],
                recv_sem=scat_recv_sems.at[s],
                device_id=(p,),
                device_id_type=pl.DeviceIdType.MESH,
            )

        def relay_fwd(s):
            for p in (6, 5):
                if (p, s) not in RELAY_IDX:
                    continue
                pltpu.make_async_remote_copy(
                    src_ref=relay_buf.at[RELAY_IDX[(p, s)]],
                    dst_ref=relay_buf.at[RELAY_IDX[(p, s)]],
                    send_sem=fwd_send_sems.at[RELAY_IDX[(p, s)]],
                    recv_sem=relay_recv_sems.at[RELAY_IDX[(p, s)]],
                    device_id=(0,), device_id_type=pl.DeviceIdType.MESH,
                ).wait_recv()
                fwd_rdma(p, s).start()

        def kv_wait(s):
            pltpu.make_async_remote_copy(
                src_ref=kv_buf.at[s], dst_ref=kv_buf.at[s],
                send_sem=scat_send_sems.at[0, s],
                recv_sem=scat_recv_sems.at[s],
                device_id=(0,), device_id_type=pl.DeviceIdType.MESH,
            ).wait_recv()

        @pl.when(my_pos == 0)
        def _():
            cps = [
                pltpu.make_async_copy(k_ref, kv_f32.at[0], local_sems.at[0]),
                pltpu.make_async_copy(v_ref, kv_f32.at[1], local_sems.at[1]),
            ]
            for cp in cps:
                cp.start()
            cps[0].wait()
            for kv, half in SLOTS:
                s = slot_id(kv, half)
                if kv == 1 and half == 0:
                    cps[1].wait()
                lo_c = half * HALF
                for p in range(1, N_DEV):
                    lo = HQ_LOC * p
                    stage_ref[p - 1, s] = kv_f32[
                        kv, :, lo:lo + HQ_LOC, :, lo_c:lo_c + HALF].astype(
                        jnp.bfloat16)
                if kv == 0 and half == 0:
                    pl.semaphore_wait(barrier_sem, N_DEV - 1)
                for r in scatter_rdmas(s):
                    r.start()
            for kv, half in SLOTS:
                s = slot_id(kv, half)
                lo_c = half * HALF
                kv_buf[s] = kv_f32[kv, :, 0:HQ_LOC, :, lo_c:lo_c + HALF].astype(
                    jnp.bfloat16)

        for cp in in_cps:
            cp.wait()
        q_all = [jnp.dot(x_v[b], wq_v[...],
                         preferred_element_type=jnp.float32) for b in range(B)]

        row = lax.broadcasted_iota(jnp.int32, (SQ, SKV), 0) // BLK
        col = lax.broadcasted_iota(jnp.int32, (SQ, SKV), 1) // BLK
        mask = col <= row

        def q_slice(b, h, c):
            return q_all[b][c * HALF:(c + 1) * HALF,
                            h * DH:(h + 1) * DH].astype(jnp.bfloat16)

        def softmax_rows(s, msk):
            s = jnp.where(msk, s, -1e9)
            m = jnp.max(s, axis=1, keepdims=True)
            w = jnp.exp(s - m)
            return (w / jnp.sum(w, axis=1, keepdims=True)).astype(jnp.bfloat16)

        @pl.when(my_pos == 4)
        def _():
            relay_fwd(slot_id(0, 0))

        @pl.when(my_pos != 0)
        def _():
            kv_wait(slot_id(0, 0))

        for b in range(B):
            for h in range(HQ_LOC):
                s = lax.dot_general(
                    q_slice(b, h, 0), kv_buf[slot_id(0, 0), b, h],
                    (((1,), (0,)), ((), ())),
                    preferred_element_type=jnp.float32) * 0.125
                w_buf[b, h, 0:HALF, 0:HALF] = softmax_rows(
                    s, mask[0:HALF, 0:HALF])

        @pl.when(my_pos == 4)
        def _():
            relay_fwd(slot_id(1, 0))

        @pl.when(my_pos != 0)
        def _():
            kv_wait(slot_id(1, 0))

        for b in range(B):
            for h in range(HQ_LOC):
                ctx_buf[b, 0:HALF, h * DH:(h + 1) * DH] = lax.dot_general(
                    w_buf[b, h, 0:HALF, 0:HALF], kv_buf[slot_id(1, 0), b, h],
                    (((1,), (1,)), ((), ())),
                    preferred_element_type=jnp.float32)

        def proj_chunk(c):
            for b in range(B):
                acc[b, c * HALF:(c + 1) * HALF, :] = jnp.dot(
                    ctx_buf[b, c * HALF:(c + 1) * HALF, :], wo_v[...],
                    preferred_element_type=jnp.float32)

        def bfly_send(s_i, c):
            partner = my_pos ^ BFLY_BITS[s_i]
            bfly_snd[c] = acc[:, c * HALF:(c + 1) * HALF, :].astype(jnp.bfloat16)
            r = pltpu.make_async_remote_copy(
                src_ref=bfly_snd.at[c],
                dst_ref=bfly_buf.at[s_i, c],
                send_sem=bfly_send_sems.at[s_i, c],
                recv_sem=bfly_recv_sems.at[s_i, c],
                device_id=(partner,),
                device_id_type=pl.DeviceIdType.MESH,
            )
            r.start()
            return r

        def bfly_finish(r, s_i, c):
            r.wait()
            acc[:, c * HALF:(c + 1) * HALF, :] = (
                acc[:, c * HALF:(c + 1) * HALF, :]
                + bfly_buf[s_i, c].astype(jnp.float32))

        inflight = [None, None]
        proj_chunk(0)
        inflight[0] = bfly_send(0, 0)

        @pl.when(my_pos == 4)
        def _():
            relay_fwd(slot_id(0, 1))

        @pl.when(my_pos != 0)
        def _():
            kv_wait(slot_id(0, 1))

        for b in range(B):
            for h in range(HQ_LOC):
                sa = lax.dot_general(
                    q_slice(b, h, 1), kv_buf[slot_id(0, 0), b, h],
                    (((1,), (0,)), ((), ())),
                    preferred_element_type=jnp.float32)
                sb = lax.dot_general(
                    q_slice(b, h, 1), kv_buf[slot_id(0, 1), b, h],
                    (((1,), (0,)), ((), ())),
                    preferred_element_type=jnp.float32)
                s = jnp.concatenate([sa, sb], axis=1) * 0.125
                w_buf[b, h, HALF:SQ, :] = softmax_rows(s, mask[HALF:SQ, :])

        @pl.when(my_pos == 4)
        def _():
            relay_fwd(slot_id(1, 1))

        @pl.when(my_pos != 0)
        def _():
            kv_wait(slot_id(1, 1))

        for b in range(B):
            for h in range(HQ_LOC):
                ctx_buf[b, HALF:SQ, h * DH:(h + 1) * DH] = (
                    lax.dot_general(
                        w_buf[b, h, HALF:SQ, 0:HALF],
                        kv_buf[slot_id(1, 0), b, h],
                        (((1,), (1,)), ((), ())),
                        preferred_element_type=jnp.float32)
                    + lax.dot_general(
                        w_buf[b, h, HALF:SQ, HALF:SKV],
                        kv_buf[slot_id(1, 1), b, h],
                        (((1,), (1,)), ((), ())),
                        preferred_element_type=jnp.float32))

        proj_chunk(1)
        inflight[1] = bfly_send(0, 1)
        for s_i in range(3):
            for c in range(2):
                bfly_finish(inflight[c], s_i, c)
                if s_i < 2:
                    inflight[c] = bfly_send(s_i + 1, c)

        out_cp = pltpu.make_async_copy(acc, out_ref, local_sems.at[5])
        out_cp.start()
        out_cp.wait()

        @pl.when(my_pos == 0)
        def _():
            for kv, half in SLOTS:
                for r in scatter_rdmas(slot_id(kv, half)):
                    r.wait_send()

        @pl.when(my_pos == 4)
        def _():
            for (p, s) in RELAY_IDX:
                fwd_rdma(p, s).wait_send()

    hbm = pl.BlockSpec(memory_space=pltpu.MemorySpace.HBM)
    return pl.pallas_call(
        body,
        out_shape=jax.ShapeDtypeStruct((B, SQ, D_MODEL), jnp.float32),
        in_specs=[hbm] * 5,
        out_specs=hbm,
        scratch_shapes=[
            pltpu.VMEM((B, SQ, D_MODEL), jnp.float32),
            pltpu.VMEM((D_MODEL, HQ_LOC * DH), jnp.float32),
            pltpu.VMEM((HQ_LOC * DH, D_MODEL), jnp.float32),
            pltpu.VMEM((B, SQ, D_MODEL), jnp.float32),
            pltpu.VMEM((4, B, HQ_LOC, DH, HALF), jnp.bfloat16),
            pltpu.VMEM((2, B, N_DEV * HQ_LOC, DH, SKV), jnp.float32),
            pltpu.VMEM((N_DEV - 1, 4, B, HQ_LOC, DH, HALF), jnp.bfloat16),
            pltpu.VMEM((6, B, HQ_LOC, DH, HALF), jnp.bfloat16),
            pltpu.VMEM((B, HQ_LOC, SQ, SKV), jnp.bfloat16),
            pltpu.VMEM((B, SQ, HQ_LOC * DH), jnp.float32),
            pltpu.VMEM((2, B, SQ // 2, D_MODEL), jnp.bfloat16),
            pltpu.VMEM((3, 2, B, SQ // 2, D_MODEL), jnp.bfloat16),
            pltpu.SemaphoreType.DMA((N_DEV - 1, 4)),
            pltpu.SemaphoreType.DMA((4,)),
            pltpu.SemaphoreType.DMA((3, 2)),
            pltpu.SemaphoreType.DMA((3, 2)),
            pltpu.SemaphoreType.DMA((6,)),
            pltpu.SemaphoreType.DMA((6,)),
            pltpu.SemaphoreType.DMA((6,)),
        ],
        compiler_params=pltpu.CompilerParams(collective_id=0),
    )(x, Wq, K_t, V_t, Wo)
